# baseline (speedup 1.0000x reference)
import functools

import jax
import jax.numpy as jnp
from jax import lax
from jax.experimental import pallas as pl
from jax.experimental.pallas import tpu as pltpu
from jax.experimental.pallas import tpu_sc as plsc

_VOCAB = 1000000
_DIM = 64
_B = 4096
_L = 200
_N = _B * _L
_NC = 2
_NS = 16
_NW = _NC * _NS
_PER_W = _N // _NW
_CHUNK = 800
_NCH = _PER_W // _CHUNK


@jax.jit
def _embed(idx, table):
  mesh = plsc.VectorSubcoreMesh(core_axis_name="c", subcore_axis_name="s")

  @functools.partial(
      pl.kernel,
      mesh=mesh,
      compiler_params=pltpu.CompilerParams(use_tc_tiling_on_sc=False),
      out_type=jax.ShapeDtypeStruct((_N, _DIM), jnp.float32),
      scratch_types=[
          pltpu.VMEM((_PER_W,), jnp.int32),
          pltpu.VMEM((_CHUNK, _DIM), jnp.float32),
          pltpu.SemaphoreType.DMA,
      ],
  )
  def k(table_hbm, idx_hbm, out_hbm, idx_v, rows_v, sem):
    wid = lax.axis_index("s") * _NC + lax.axis_index("c")
    base = wid * _PER_W
    pltpu.sync_copy(idx_hbm.at[pl.ds(base, _PER_W)], idx_v)

    def body(c, carry):
      off = c * _CHUNK
      pltpu.async_copy(
          table_hbm.at[idx_v.at[pl.ds(off, _CHUNK)]], rows_v, sem
      ).wait()
      pltpu.sync_copy(rows_v, out_hbm.at[pl.ds(base + off, _CHUNK)])
      return carry

    lax.fori_loop(0, _NCH, body, 0)

  return k(table, idx)


def kernel(x, table):
  out = _embed(x.reshape(_N), table)
  return out.reshape(_B, _L, _DIM)


# speedup vs baseline: 1.0061x; 1.0061x over previous
import functools

import jax
import jax.numpy as jnp
from jax import lax
from jax.experimental import pallas as pl
from jax.experimental.pallas import tpu as pltpu
from jax.experimental.pallas import tpu_sc as plsc

_VOCAB = 1000000
_DIM = 64
_B = 4096
_L = 200
_N = _B * _L
_NC = 2
_NS = 16
_NW = _NC * _NS
_PER_W = _N // _NW
_CHUNK = 800
_NCH = _PER_W // _CHUNK


@jax.jit
def _embed(idx, table):
  mesh = plsc.VectorSubcoreMesh(core_axis_name="c", subcore_axis_name="s")

  @functools.partial(
      pl.kernel,
      mesh=mesh,
      compiler_params=pltpu.CompilerParams(use_tc_tiling_on_sc=False),
      out_type=jax.ShapeDtypeStruct((_N, _DIM), jnp.float32),
      scratch_types=[
          pltpu.VMEM((_PER_W,), jnp.int32),
          pltpu.VMEM((_CHUNK, _DIM), jnp.float32),
          pltpu.VMEM((_CHUNK, _DIM), jnp.float32),
          pltpu.SemaphoreType.DMA,
          pltpu.SemaphoreType.DMA,
          pltpu.SemaphoreType.DMA,
          pltpu.SemaphoreType.DMA,
      ],
  )
  def k(table_hbm, idx_hbm, out_hbm, idx_v, rows0, rows1, g0, g1, s0, s1):
    wid = lax.axis_index("s") * _NC + lax.axis_index("c")
    base = wid * _PER_W
    pltpu.sync_copy(idx_hbm.at[pl.ds(base, _PER_W)], idx_v)
    rows = (rows0, rows1)
    gsem = (g0, g1)
    ssem = (s0, s1)

    def gather(c, b):
      pltpu.async_copy(
          table_hbm.at[idx_v.at[pl.ds(c * _CHUNK, _CHUNK)]], rows[b], gsem[b]
      )

    def store(c, b):
      pltpu.async_copy(
          rows[b], out_hbm.at[pl.ds(base + c * _CHUNK, _CHUNK)], ssem[b]
      )

    def wait_gather(b):
      pltpu.make_async_copy(
          table_hbm.at[idx_v.at[pl.ds(0, _CHUNK)]], rows[b], gsem[b]
      ).wait()

    def wait_store(b):
      pltpu.make_async_copy(
          rows[b], out_hbm.at[pl.ds(base, _CHUNK)], ssem[b]
      ).wait()

    for b in range(2):
      gather(b, b)

    def group(g, carry):
      c0 = g * 2
      for b in range(2):
        wait_gather(b)
        store(c0 + b, b)
      for b in range(2):
        wait_store(b)
        nxt = c0 + b + 2

        @pl.when(nxt < _NCH)
        def _():
          gather(nxt, b)

      return carry

    lax.fori_loop(0, _NCH // 2, group, 0)

  return k(table, idx)


def kernel(x, table):
  out = _embed(x.reshape(_N), table)
  return out.reshape(_B, _L, _DIM)
